# baseline (speedup 1.0000x reference)
import functools

import numpy as np
import jax
import jax.numpy as jnp
from jax import lax
from jax.experimental import pallas as pl
from jax.experimental.pallas import tpu as pltpu
from jax.experimental.pallas import tpu_sc as plsc

_B, _H, _QQ, _KK = 8, 8, 1024, 1024
_G = 32


def _make_w_idx() -> np.ndarray:
    qx = np.arange(_G)
    kx = np.arange(_G)
    absm = np.abs(kx[None, :] - qx[:, None])
    idx = (np.arange(_H)[:, None, None, None] * _KK
           + _G * np.arange(_G)[None, :, None, None]
           + absm[None, None, :, :]).astype(np.int32)
    return idx.reshape(-1)


_W_IDX = _make_w_idx()


def _sc_gather_w(bbT_flat, idx_flat):
    info = plsc.get_sparse_core_info()
    nc, ns, lanes = info.num_cores, info.num_subcores, info.num_lanes
    nw = nc * ns
    n = idx_flat.shape[0]
    chunk = n // nw
    tbl = bbT_flat.shape[0]
    mesh = plsc.VectorSubcoreMesh(core_axis_name="c", subcore_axis_name="s")

    @functools.partial(
        pl.kernel, mesh=mesh,
        out_type=jax.ShapeDtypeStruct((n,), jnp.float32),
        compiler_params=pltpu.CompilerParams(needs_layout_passes=False),
        scratch_types=[
            pltpu.VMEM((tbl,), jnp.float32),
            pltpu.VMEM((chunk,), jnp.int32),
            pltpu.VMEM((chunk,), jnp.float32),
        ],
    )
    def k(table_hbm, idx_hbm, out_hbm, table_v, idx_v, out_v):
        wid = lax.axis_index("s") * nc + lax.axis_index("c")
        base = wid * chunk
        pltpu.sync_copy(table_hbm, table_v)
        pltpu.sync_copy(idx_hbm.at[pl.ds(base, chunk)], idx_v)

        def body(i, carry):
            iv = idx_v[pl.ds(i * lanes, lanes)]
            out_v[pl.ds(i * lanes, lanes)] = plsc.load_gather(table_v, [iv])
            return carry

        lax.fori_loop(0, chunk // lanes, body, 0)
        pltpu.sync_copy(out_v, out_hbm.at[pl.ds(base, chunk)])

    return k(bbT_flat, idx_flat)


def _add_body(w_ref, in_ref, out_ref, pos_ref):
    b = pl.program_id(1)

    @pl.when(b == 0)
    def _build_bias():
        for ky in range(_G):
            pos_ref[0:_G, ky * _G:(ky + 1) * _G] = w_ref[0, ky]
        for qy in range(1, _G):
            r = qy * _G
            pos_ref[r:r + _G, _G:] = pos_ref[r - _G:r, :_KK - _G]
            pos_ref[r:r + _G, 0:_G] = w_ref[0, qy]

    pos = pos_ref[...]
    out_ref[0, 0] = in_ref[0, 0] + pos
    out_ref[1, 0] = in_ref[1, 0] + pos


@jax.jit
def kernel(inputs, bb):
    bbT_flat = jnp.transpose(bb).reshape(-1)
    w_flat = _sc_gather_w(bbT_flat, jnp.asarray(_W_IDX))
    w4 = w_flat.reshape(_H, _G, _G, _G)

    return pl.pallas_call(
        _add_body,
        grid=(_H, _B // 2),
        in_specs=[
            pl.BlockSpec((1, _G, _G, _G), lambda h, b: (h, 0, 0, 0)),
            pl.BlockSpec((2, 1, _QQ, _KK), lambda h, b: (b, h, 0, 0)),
        ],
        out_specs=pl.BlockSpec((2, 1, _QQ, _KK), lambda h, b: (b, h, 0, 0)),
        out_shape=jax.ShapeDtypeStruct((_B, _H, _QQ, _KK), jnp.float32),
        scratch_shapes=[pltpu.VMEM((_QQ, _KK), jnp.float32)],
    )(w4, inputs)


# speedup vs baseline: 1.0110x; 1.0110x over previous
import functools

import numpy as np
import jax
import jax.numpy as jnp
from jax import lax
from jax.experimental import pallas as pl
from jax.experimental.pallas import tpu as pltpu
from jax.experimental.pallas import tpu_sc as plsc

_B, _H, _QQ, _KK = 8, 8, 1024, 1024
_G = 32


def _make_w_idx() -> np.ndarray:
    qx = np.arange(_G)
    kx = np.arange(_G)
    absm = np.abs(kx[None, :] - qx[:, None])
    idx = (np.arange(_H)[:, None, None, None] * _KK
           + _G * np.arange(_G)[None, :, None, None]
           + absm[None, None, :, :]).astype(np.int32)
    return idx.reshape(-1)


_W_IDX = _make_w_idx()


def _sc_gather_w(bbT_flat, idx_flat):
    info = plsc.get_sparse_core_info()
    nc, ns, lanes = info.num_cores, info.num_subcores, info.num_lanes
    nw = nc * ns
    n = idx_flat.shape[0]
    chunk = n // nw
    tbl = bbT_flat.shape[0]
    mesh = plsc.VectorSubcoreMesh(core_axis_name="c", subcore_axis_name="s")

    @functools.partial(
        pl.kernel, mesh=mesh,
        out_type=jax.ShapeDtypeStruct((n,), jnp.float32),
        compiler_params=pltpu.CompilerParams(needs_layout_passes=False),
        scratch_types=[
            pltpu.VMEM((tbl,), jnp.float32),
            pltpu.VMEM((chunk,), jnp.int32),
            pltpu.VMEM((chunk,), jnp.float32),
        ],
    )
    def k(table_hbm, idx_hbm, out_hbm, table_v, idx_v, out_v):
        wid = lax.axis_index("s") * nc + lax.axis_index("c")
        base = wid * chunk
        pltpu.sync_copy(table_hbm, table_v)
        pltpu.sync_copy(idx_hbm.at[pl.ds(base, chunk)], idx_v)

        @plsc.parallel_loop(0, chunk // lanes, unroll=8)
        def body(i):
            iv = idx_v[pl.ds(i * lanes, lanes)]
            out_v[pl.ds(i * lanes, lanes)] = plsc.load_gather(table_v, [iv])
        pltpu.sync_copy(out_v, out_hbm.at[pl.ds(base, chunk)])

    return k(bbT_flat, idx_flat)


def _add_body(w_ref, in_ref, out_ref, pos_ref):
    b = pl.program_id(1)

    @pl.when(b == 0)
    def _build_bias():
        for ky in range(_G):
            pos_ref[0:_G, ky * _G:(ky + 1) * _G] = w_ref[0, ky]
        for qy in range(1, _G):
            r = qy * _G
            pos_ref[r:r + _G, _G:] = pos_ref[r - _G:r, :_KK - _G]
            pos_ref[r:r + _G, 0:_G] = w_ref[0, qy]

    pos = pos_ref[...]
    out_ref[0, 0] = in_ref[0, 0] + pos
    out_ref[1, 0] = in_ref[1, 0] + pos


@jax.jit
def kernel(inputs, bb):
    bbT_flat = jnp.transpose(bb).reshape(-1)
    w_flat = _sc_gather_w(bbT_flat, jnp.asarray(_W_IDX))
    w4 = w_flat.reshape(_H, _G, _G, _G)

    return pl.pallas_call(
        _add_body,
        grid=(_H, _B // 2),
        in_specs=[
            pl.BlockSpec((1, _G, _G, _G), lambda h, b: (h, 0, 0, 0)),
            pl.BlockSpec((2, 1, _QQ, _KK), lambda h, b: (b, h, 0, 0)),
        ],
        out_specs=pl.BlockSpec((2, 1, _QQ, _KK), lambda h, b: (b, h, 0, 0)),
        out_shape=jax.ShapeDtypeStruct((_B, _H, _QQ, _KK), jnp.float32),
        scratch_shapes=[pltpu.VMEM((_QQ, _KK), jnp.float32)],
    )(w4, inputs)


# speedup vs baseline: 1.0268x; 1.0156x over previous
import functools

import numpy as np
import jax
import jax.numpy as jnp
from jax import lax
from jax.experimental import pallas as pl
from jax.experimental.pallas import tpu as pltpu
from jax.experimental.pallas import tpu_sc as plsc

_B, _H, _QQ, _KK = 8, 8, 1024, 1024
_G = 32


def _make_w_idx() -> np.ndarray:
    qx = np.arange(_G)
    kx = np.arange(_G)
    absm = np.abs(kx[None, :] - qx[:, None])
    idx = (_G * np.arange(_G)[:, None, None] + absm[None, :, :]).astype(np.int32)
    return idx.reshape(-1)


_W_IDX = _make_w_idx()


def _sc_gather_w(bbT_flat, idx_flat):
    info = plsc.get_sparse_core_info()
    nc, ns, lanes = info.num_cores, info.num_subcores, info.num_lanes
    nw = nc * ns
    n = _H * idx_flat.shape[0]
    chunk = n // nw
    tiles_per_head = (nw // _H)
    mesh = plsc.VectorSubcoreMesh(core_axis_name="c", subcore_axis_name="s")

    @functools.partial(
        pl.kernel, mesh=mesh,
        out_type=jax.ShapeDtypeStruct((n,), jnp.float32),
        compiler_params=pltpu.CompilerParams(needs_layout_passes=False),
        scratch_types=[
            pltpu.VMEM((_KK,), jnp.float32),
            pltpu.VMEM((chunk,), jnp.int32),
            pltpu.VMEM((chunk,), jnp.float32),
            pltpu.SemaphoreType.DMA,
            pltpu.SemaphoreType.DMA,
        ],
    )
    def k(table_hbm, idx_hbm, out_hbm, table_v, idx_v, out_v, sem_t, sem_i):
        wid = lax.axis_index("s") * nc + lax.axis_index("c")
        h = wid // tiles_per_head
        quarter = wid % tiles_per_head
        ct = pltpu.async_copy(table_hbm.at[pl.ds(h * _KK, _KK)], table_v, sem_t)
        ci = pltpu.async_copy(idx_hbm.at[pl.ds(quarter * chunk, chunk)], idx_v, sem_i)
        ct.wait()
        ci.wait()

        @plsc.parallel_loop(0, chunk // lanes, unroll=8)
        def body(i):
            iv = idx_v[pl.ds(i * lanes, lanes)]
            out_v[pl.ds(i * lanes, lanes)] = plsc.load_gather(table_v, [iv])
        pltpu.sync_copy(out_v, out_hbm.at[pl.ds(wid * chunk, chunk)])

    return k(bbT_flat, idx_flat)


def _add_body(w_ref, in_ref, out_ref, pos_ref):
    b = pl.program_id(1)

    @pl.when(b == 0)
    def _build_bias():
        for ky in range(_G):
            pos_ref[0:_G, ky * _G:(ky + 1) * _G] = w_ref[0, ky]
        for qy in range(1, _G):
            r = qy * _G
            pos_ref[r:r + _G, _G:] = pos_ref[r - _G:r, :_KK - _G]
            pos_ref[r:r + _G, 0:_G] = w_ref[0, qy]

    pos = pos_ref[...]
    out_ref[0, 0] = in_ref[0, 0] + pos
    out_ref[1, 0] = in_ref[1, 0] + pos


@jax.jit
def kernel(inputs, bb):
    bbT_flat = jnp.transpose(bb).reshape(-1)
    w_flat = _sc_gather_w(bbT_flat, jnp.asarray(_W_IDX))
    w4 = w_flat.reshape(_H, _G, _G, _G)

    return pl.pallas_call(
        _add_body,
        grid=(_H, _B // 2),
        in_specs=[
            pl.BlockSpec((1, _G, _G, _G), lambda h, b: (h, 0, 0, 0)),
            pl.BlockSpec((2, 1, _QQ, _KK), lambda h, b: (b, h, 0, 0)),
        ],
        out_specs=pl.BlockSpec((2, 1, _QQ, _KK), lambda h, b: (b, h, 0, 0)),
        out_shape=jax.ShapeDtypeStruct((_B, _H, _QQ, _KK), jnp.float32),
        scratch_shapes=[pltpu.VMEM((_QQ, _KK), jnp.float32)],
    )(w4, inputs)
